# Initial kernel scaffold; baseline (speedup 1.0000x reference)
import jax
import jax.numpy as jnp
from jax.experimental import pallas as pl


def kernel(time_steps, table):
    raise NotImplementedError("write your pallas kernel here")


# speedup vs baseline: 4.4420x; 4.4420x over previous
import functools

import jax
import jax.numpy as jnp
from jax import lax
from jax.experimental import pallas as pl
from jax.experimental.pallas import tpu as pltpu
from jax.experimental.pallas import tpu_sc as plsc

_D = 32
_L = 128
_NC = 2
_NS = 16
_NW = _NC * _NS


def _gather_body(idx_hbm, table_hbm, out_hbm, idx_v, rows_v, sem):
    wid = lax.axis_index("s") * _NC + lax.axis_index("c")
    b_per_w = idx_hbm.shape[0] // _NW
    n_chunks = b_per_w // _L
    base = wid * b_per_w

    def body(j, carry):
        off = base + j * _L
        pltpu.sync_copy(idx_hbm.at[pl.ds(off, _L)], idx_v)
        pltpu.async_copy(table_hbm.at[idx_v], rows_v, sem).wait()
        pltpu.sync_copy(rows_v, out_hbm.at[pl.ds(off, _L)])
        return carry

    lax.fori_loop(0, n_chunks, body, 0)


@jax.jit
def _run(idx_flat, table):
    b = idx_flat.shape[0]
    mesh = plsc.VectorSubcoreMesh(core_axis_name="c", subcore_axis_name="s")
    return pl.kernel(
        _gather_body,
        mesh=mesh,
        out_type=jax.ShapeDtypeStruct((b, _D), jnp.float32),
        scratch_types=[
            pltpu.VMEM((_L,), jnp.int32),
            pltpu.VMEM((_L, _D), jnp.float32),
            pltpu.SemaphoreType.DMA,
        ],
        compiler_params=pltpu.CompilerParams(use_tc_tiling_on_sc=False),
    )(idx_flat, table)


def kernel(time_steps, table):
    s0, s1 = time_steps.shape
    idx_flat = time_steps.reshape(-1).astype(jnp.int32)
    out = _run(idx_flat, table)
    return out.reshape(s0, s1, _D)


# speedup vs baseline: 6.4809x; 1.4590x over previous
import functools

import jax
import jax.numpy as jnp
from jax import lax
from jax.experimental import pallas as pl
from jax.experimental.pallas import tpu as pltpu
from jax.experimental.pallas import tpu_sc as plsc

_D = 32
_L = 128
_NC = 2
_NS = 16
_NW = _NC * _NS
_CH = 1024
_NG = _CH // _L
_NBUF = 2


def _gather_body(idx_hbm, table_hbm, out_hbm, *scratch):
    idx_v, rows_v = scratch[0], scratch[1]
    sem_idx = scratch[2:2 + _NBUF]
    sem_gat = scratch[2 + _NBUF:2 + 2 * _NBUF]
    sem_out = scratch[2 + 2 * _NBUF:2 + 3 * _NBUF]

    wid = lax.axis_index("s") * _NC + lax.axis_index("c")
    b_per_w = out_hbm.shape[0] // _NW
    n_chunks = b_per_w // _CH
    base = wid * b_per_w
    rbase = base // _L

    def start_idx(c, b):
        pltpu.async_copy(
            idx_hbm.at[pl.ds(rbase + c * _NG, _NG)], idx_v.at[b], sem_idx[b])

    def wait_idx(b):
        pltpu.make_async_copy(
            idx_hbm.at[pl.ds(0, _NG)], idx_v.at[b], sem_idx[b]).wait()

    def wait_gat(b):
        pltpu.make_async_copy(
            table_hbm.at[pl.ds(0, _CH)], rows_v.at[b], sem_gat[b]).wait()

    def start_out(c, b):
        pltpu.async_copy(
            rows_v.at[b], out_hbm.at[pl.ds(base + c * _CH, _CH)], sem_out[b])

    def wait_out(b):
        pltpu.make_async_copy(
            rows_v.at[b], out_hbm.at[pl.ds(0, _CH)], sem_out[b]).wait()

    for b in range(_NBUF):
        start_idx(b, b)

    def outer(g, carry):
        for b in range(_NBUF):
            c = g * _NBUF + b
            wait_idx(b)

            @pl.when(g > 0)
            def _():
                wait_out(b)

            for r in range(_NG):
                pltpu.async_copy(
                    table_hbm.at[idx_v.at[b, r]],
                    rows_v.at[b, pl.ds(r * _L, _L)],
                    sem_gat[b])
            wait_gat(b)
            start_out(c, b)

            @pl.when(c + _NBUF < n_chunks)
            def _():
                start_idx(c + _NBUF, b)
        return carry

    lax.fori_loop(0, n_chunks // _NBUF, outer, 0)

    for b in range(_NBUF):
        wait_out(b)


@jax.jit
def _run(idx2d, table):
    b = idx2d.shape[0] * _L
    mesh = plsc.VectorSubcoreMesh(core_axis_name="c", subcore_axis_name="s")
    scratch = [
        pltpu.VMEM((_NBUF, _NG, _L), jnp.int32),
        pltpu.VMEM((_NBUF, _CH, _D), jnp.float32),
    ] + [pltpu.SemaphoreType.DMA] * (3 * _NBUF)
    return pl.kernel(
        _gather_body,
        mesh=mesh,
        out_type=jax.ShapeDtypeStruct((b, _D), jnp.float32),
        scratch_types=scratch,
        compiler_params=pltpu.CompilerParams(use_tc_tiling_on_sc=False),
    )(idx2d, table)


def kernel(time_steps, table):
    s0, s1 = time_steps.shape
    idx2d = time_steps.reshape(-1, _L).astype(jnp.int32)
    out = _run(idx2d, table)
    return out.reshape(s0, s1, _D)


# speedup vs baseline: 6.5073x; 1.0041x over previous
import functools

import jax
import jax.numpy as jnp
from jax import lax
from jax.experimental import pallas as pl
from jax.experimental.pallas import tpu as pltpu
from jax.experimental.pallas import tpu_sc as plsc

_D = 32
_L = 128
_NC = 2
_NS = 16
_NW = _NC * _NS
_CH = 512
_NG = _CH // _L
_NBUF = 4


def _gather_body(idx_hbm, table_hbm, out_hbm, *scratch):
    idx_v, rows_v = scratch[0], scratch[1]
    sem_idx = scratch[2:2 + _NBUF]
    sem_gat = scratch[2 + _NBUF:2 + 2 * _NBUF]
    sem_out = scratch[2 + 2 * _NBUF:2 + 3 * _NBUF]

    wid = lax.axis_index("s") * _NC + lax.axis_index("c")
    b_per_w = out_hbm.shape[0] // _NW
    n_chunks = b_per_w // _CH
    base = wid * b_per_w
    rbase = base // _L

    def start_idx(c, b):
        pltpu.async_copy(
            idx_hbm.at[pl.ds(rbase + c * _NG, _NG)], idx_v.at[b], sem_idx[b])

    def wait_idx(b):
        pltpu.make_async_copy(
            idx_hbm.at[pl.ds(0, _NG)], idx_v.at[b], sem_idx[b]).wait()

    def wait_gat(b):
        pltpu.make_async_copy(
            table_hbm.at[pl.ds(0, _CH)], rows_v.at[b], sem_gat[b]).wait()

    def start_out(c, b):
        pltpu.async_copy(
            rows_v.at[b], out_hbm.at[pl.ds(base + c * _CH, _CH)], sem_out[b])

    def wait_out(b):
        pltpu.make_async_copy(
            rows_v.at[b], out_hbm.at[pl.ds(0, _CH)], sem_out[b]).wait()

    for b in range(_NBUF):
        start_idx(b, b)

    def fire_gathers(b):
        for r in range(_NG):
            pltpu.async_copy(
                table_hbm.at[idx_v.at[b, r]],
                rows_v.at[b, pl.ds(r * _L, _L)],
                sem_gat[b])

    def outer(g, carry):
        for b in range(_NBUF):
            c = g * _NBUF + b
            wait_idx(b)

            @pl.when(g > 0)
            def _():
                wait_out(b)

            fire_gathers(b)

            bp = (b - 1) % _NBUF

            @pl.when(c > 0)
            def _():
                wait_gat(bp)
                start_out(c - 1, bp)

                @pl.when(c - 1 + _NBUF < n_chunks)
                def _():
                    start_idx(c - 1 + _NBUF, bp)
        return carry

    lax.fori_loop(0, n_chunks // _NBUF, outer, 0)

    bl = (n_chunks - 1) % _NBUF
    wait_gat(bl)
    start_out(n_chunks - 1, bl)
    for b in range(_NBUF):
        wait_out(b)


@jax.jit
def _run(idx2d, table):
    b = idx2d.shape[0] * _L
    mesh = plsc.VectorSubcoreMesh(core_axis_name="c", subcore_axis_name="s")
    scratch = [
        pltpu.VMEM((_NBUF, _NG, _L), jnp.int32),
        pltpu.VMEM((_NBUF, _CH, _D), jnp.float32),
    ] + [pltpu.SemaphoreType.DMA] * (3 * _NBUF)
    return pl.kernel(
        _gather_body,
        mesh=mesh,
        out_type=jax.ShapeDtypeStruct((b, _D), jnp.float32),
        scratch_types=scratch,
        compiler_params=pltpu.CompilerParams(use_tc_tiling_on_sc=False),
    )(idx2d, table)


def kernel(time_steps, table):
    s0, s1 = time_steps.shape
    idx2d = time_steps.reshape(-1, _L).astype(jnp.int32)
    out = _run(idx2d, table)
    return out.reshape(s0, s1, _D)
